# baseline (speedup 1.0000x reference)
import functools

import jax
import jax.numpy as jnp
from jax import lax
from jax.experimental import pallas as pl
from jax.experimental.pallas import tpu as pltpu
from jax.experimental.pallas import tpu_sc as plsc

_B, _L, _D = 64, 2048, 300
_DP = 320
_N = _B * _L
_NC, _NS = 2, 16
_NW = _NC * _NS
_TPW = _N // _NW
_C = 64
_NCHUNK = _TPW // _C
_NBLK = _N // _C
_LANES = 16
_NGRP = 19

_mesh = plsc.VectorSubcoreMesh(core_axis_name="c", subcore_axis_name="s")


@functools.partial(
    pl.kernel,
    out_type=jax.ShapeDtypeStruct((_N, _DP), jnp.bfloat16),
    mesh=_mesh,
    compiler_params=pltpu.CompilerParams(use_tc_tiling_on_sc=False, needs_layout_passes=False),
    scratch_types=[
        pltpu.VMEM((4, 5, _C), jnp.int32),
        pltpu.VMEM((3, _C, _DP), jnp.bfloat16),
        pltpu.SemaphoreType.DMA,
        pltpu.SemaphoreType.DMA,
        pltpu.SemaphoreType.DMA,
        pltpu.SemaphoreType.DMA,
        pltpu.SemaphoreType.DMA,
        pltpu.SemaphoreType.DMA,
        pltpu.SemaphoreType.DMA,
        pltpu.SemaphoreType.DMA,
        pltpu.SemaphoreType.DMA,
        pltpu.SemaphoreType.DMA,
        pltpu.SemaphoreType.DMA,
        pltpu.SemaphoreType.DMA,
        pltpu.SemaphoreType.DMA,
    ],
)
def _embed_sum(idx_hbm,
               pos_hbm, col_hbm, row_hbm, rank_hbm, out_hbm,
               idxb, acc,
               isem0, isem1, isem2, isem3,
               psem0, psem1, psem2,
               asem0, asem1, asem2,
               ssem0, ssem1, ssem2):
  wid = lax.axis_index("s") * _NC + lax.axis_index("c")
  base0 = wid * _TPW
  blk0 = wid * _NCHUNK
  isems = [isem0, isem1, isem2, isem3]
  psems = [psem0, psem1, psem2]
  asems = [asem0, asem1, asem2]
  ssems = [ssem0, ssem1, ssem2]
  addtabs = [col_hbm, row_hbm, rank_hbm]

  def fire_idx(c, s4):
    pltpu.async_copy(idx_hbm.at[blk0 + c], idxb.at[s4], isems[s4])

  def wait_idx(s4):
    pltpu.make_async_copy(idx_hbm.at[blk0], idxb.at[s4], isems[s4]).wait()

  def fire_pos_tok(c, s4, s3):
    pltpu.async_copy(pos_hbm.at[idxb.at[s4, 0]], acc.at[s3], psems[s3])

  def wait_pos(s4, s3):
    pltpu.make_async_copy(pos_hbm.at[idxb.at[s4, 0]], acc.at[s3],
                          psems[s3]).wait()

  def fire_adds(s4, s3):
    for t in range(3):
      pltpu.async_copy(addtabs[t].at[idxb.at[s4, t + 2]], acc.at[s3],
                       asems[s3], add=True)

  def wait_adds_tok(s4, s3):
    for t in range(3):
      pltpu.make_async_copy(addtabs[t].at[idxb.at[s4, t + 2]], acc.at[s3],
                            asems[s3]).wait()

  def fire_store(c, s3):
    pltpu.async_copy(acc.at[s3], out_hbm.at[pl.ds(base0 + c * _C, _C)],
                     ssems[s3])

  def wait_store(s3):
    pltpu.make_async_copy(acc.at[s3], out_hbm.at[pl.ds(base0, _C)],
                          ssems[s3]).wait()

  fire_idx(0, 0)
  fire_idx(1, 1)
  fire_idx(2, 2)
  wait_idx(0)
  fire_pos_tok(0, 0, 0)
  wait_idx(1)
  fire_pos_tok(1, 1, 1)
  wait_pos(0, 0)
  fire_adds(0, 0)

  def chunk(c, s4, s3, s4n, s3n, s4nn, s3nn,
            do_idx, do_adds_next, do_loads_next2, do_store_wait):
    if do_idx:
      fire_idx(c + 3, (s4 + 3) % 4)
    if do_adds_next:
      wait_pos(s4n, s3n)
      fire_adds(s4n, s3n)
    wait_adds_tok(s4, s3)
    fire_store(c, s3)
    if do_loads_next2:
      if do_store_wait:
        wait_store(s3nn)
      wait_idx(s4nn)
      fire_pos_tok(c + 2, s4nn, s3nn)

  chunk(0, 0, 0, 1, 1, 2, 2, True, True, True, False)

  def steady12(m, carry):
    c0 = 1 + m * 12
    for k in range(12):
      ck = c0 + k
      s4 = (1 + k) % 4
      s3 = (1 + k) % 3
      chunk(ck, s4, s3, (s4 + 1) % 4, (s3 + 1) % 3, (s4 + 2) % 4,
            (s3 + 2) % 3, True, True, True, True)
    return carry

  lax.fori_loop(0, 5, steady12, 0, unroll=False)

  chunk(61, 61 % 4, 61 % 3, 62 % 4, 62 % 3, 63 % 4, 63 % 3,
        False, True, True, True)
  chunk(62, 62 % 4, 62 % 3, 63 % 4, 63 % 3, 0, 0, False, True, False, False)
  chunk(63, 63 % 4, 63 % 3, 0, 0, 0, 0, False, False, False, False)

  wait_store(61 % 3)
  wait_store(62 % 3)
  wait_store(63 % 3)


def _pad(t):
  return jnp.pad(t, ((0, 0), (0, _DP - _D))).astype(jnp.bfloat16)


_RN = 1024


def _combine_body(tok_ref, sums_ref, segf_ref, diff_ref, o_ref):
  o_ref[...] = (tok_ref[...] + sums_ref[:, :_D].astype(jnp.float32)
                + segf_ref[...] * diff_ref[...])


_combine = pl.pallas_call(
    _combine_body,
    out_shape=jax.ShapeDtypeStruct((_N, _D), jnp.float32),
    grid=(_N // _RN,),
    in_specs=[
        pl.BlockSpec((_RN, _D), lambda i: (i, 0)),
        pl.BlockSpec((_RN, _DP), lambda i: (i, 0)),
        pl.BlockSpec((_RN, 1), lambda i: (i, 0)),
        pl.BlockSpec((1, _D), lambda i: (0, 0)),
    ],
    out_specs=pl.BlockSpec((_RN, _D), lambda i: (i, 0)),
)


def kernel(token_vecs, pos_idx, seg_idx, col_idx, row_idx, rank_idx,
           pos, seg_id, col_id, row_id, rank_id):
  idx = jnp.stack([pos_idx.reshape(_N), seg_idx.reshape(_N),
                   col_idx.reshape(_N), row_idx.reshape(_N),
                   rank_idx.reshape(_N)], axis=0)
  idx_blocks = idx.reshape(5, _NBLK, _C).transpose(1, 0, 2)
  pos_eff = pos + seg_id[0][None, :]
  diff = (seg_id[1] - seg_id[0])[None, :]
  sums = _embed_sum(
      idx_blocks,
      _pad(pos_eff), _pad(col_id), _pad(row_id), _pad(rank_id))
  out = _combine(token_vecs.reshape(_N, _D), sums,
                 seg_idx.reshape(_N, 1).astype(jnp.float32), diff)
  return out.reshape(_B, _L, _D)


# speedup vs baseline: 1.0133x; 1.0133x over previous
import functools

import jax
import jax.numpy as jnp
from jax import lax
from jax.experimental import pallas as pl
from jax.experimental.pallas import tpu as pltpu
from jax.experimental.pallas import tpu_sc as plsc

_B, _L, _D = 64, 2048, 300
_DP = 304
_N = _B * _L
_NC, _NS = 2, 16
_NW = _NC * _NS
_TPW = _N // _NW
_C = 64
_NCHUNK = _TPW // _C
_NBLK = _N // _C
_LANES = 16
_NGRP = 19

_mesh = plsc.VectorSubcoreMesh(core_axis_name="c", subcore_axis_name="s")


@functools.partial(
    pl.kernel,
    out_type=jax.ShapeDtypeStruct((_N, _DP), jnp.float32),
    mesh=_mesh,
    compiler_params=pltpu.CompilerParams(use_tc_tiling_on_sc=False, needs_layout_passes=False),
    scratch_types=[
        pltpu.VMEM((4, 5, _C), jnp.int32),
        pltpu.VMEM((3, _C, _DP), jnp.float32),
        pltpu.SemaphoreType.DMA,
        pltpu.SemaphoreType.DMA,
        pltpu.SemaphoreType.DMA,
        pltpu.SemaphoreType.DMA,
        pltpu.SemaphoreType.DMA,
        pltpu.SemaphoreType.DMA,
        pltpu.SemaphoreType.DMA,
        pltpu.SemaphoreType.DMA,
        pltpu.SemaphoreType.DMA,
        pltpu.SemaphoreType.DMA,
        pltpu.SemaphoreType.DMA,
        pltpu.SemaphoreType.DMA,
        pltpu.SemaphoreType.DMA,
    ],
)
def _embed_sum(idx_hbm,
               pos_hbm, col_hbm, row_hbm, rank_hbm, out_hbm,
               idxb, acc,
               isem0, isem1, isem2, isem3,
               psem0, psem1, psem2,
               asem0, asem1, asem2,
               ssem0, ssem1, ssem2):
  wid = lax.axis_index("s") * _NC + lax.axis_index("c")
  base0 = wid * _TPW
  blk0 = wid * _NCHUNK
  isems = [isem0, isem1, isem2, isem3]
  psems = [psem0, psem1, psem2]
  asems = [asem0, asem1, asem2]
  ssems = [ssem0, ssem1, ssem2]
  addtabs = [col_hbm, row_hbm, rank_hbm]

  def fire_idx(c, s4):
    pltpu.async_copy(idx_hbm.at[blk0 + c], idxb.at[s4], isems[s4])

  def wait_idx(s4):
    pltpu.make_async_copy(idx_hbm.at[blk0], idxb.at[s4], isems[s4]).wait()

  def fire_pos_tok(c, s4, s3):
    pltpu.async_copy(pos_hbm.at[idxb.at[s4, 0]], acc.at[s3], psems[s3])

  def wait_pos(s4, s3):
    pltpu.make_async_copy(pos_hbm.at[idxb.at[s4, 0]], acc.at[s3],
                          psems[s3]).wait()

  def fire_adds(s4, s3):
    for t in range(3):
      pltpu.async_copy(addtabs[t].at[idxb.at[s4, t + 2]], acc.at[s3],
                       asems[s3], add=True)

  def wait_adds_tok(s4, s3):
    for t in range(3):
      pltpu.make_async_copy(addtabs[t].at[idxb.at[s4, t + 2]], acc.at[s3],
                            asems[s3]).wait()

  def fire_store(c, s3):
    pltpu.async_copy(acc.at[s3], out_hbm.at[pl.ds(base0 + c * _C, _C)],
                     ssems[s3])

  def wait_store(s3):
    pltpu.make_async_copy(acc.at[s3], out_hbm.at[pl.ds(base0, _C)],
                          ssems[s3]).wait()

  fire_idx(0, 0)
  fire_idx(1, 1)
  fire_idx(2, 2)
  wait_idx(0)
  fire_pos_tok(0, 0, 0)
  wait_idx(1)
  fire_pos_tok(1, 1, 1)
  wait_pos(0, 0)
  fire_adds(0, 0)

  def chunk(c, s4, s3, s4n, s3n, s4nn, s3nn,
            do_idx, do_adds_next, do_loads_next2, do_store_wait):
    if do_idx:
      fire_idx(c + 3, (s4 + 3) % 4)
    if do_adds_next:
      wait_pos(s4n, s3n)
      fire_adds(s4n, s3n)
    wait_adds_tok(s4, s3)
    fire_store(c, s3)
    if do_loads_next2:
      if do_store_wait:
        wait_store(s3nn)
      wait_idx(s4nn)
      fire_pos_tok(c + 2, s4nn, s3nn)

  chunk(0, 0, 0, 1, 1, 2, 2, True, True, True, False)

  def steady12(m, carry):
    c0 = 1 + m * 12
    for k in range(12):
      ck = c0 + k
      s4 = (1 + k) % 4
      s3 = (1 + k) % 3
      chunk(ck, s4, s3, (s4 + 1) % 4, (s3 + 1) % 3, (s4 + 2) % 4,
            (s3 + 2) % 3, True, True, True, True)
    return carry

  lax.fori_loop(0, 5, steady12, 0, unroll=False)

  chunk(61, 61 % 4, 61 % 3, 62 % 4, 62 % 3, 63 % 4, 63 % 3,
        False, True, True, True)
  chunk(62, 62 % 4, 62 % 3, 63 % 4, 63 % 3, 0, 0, False, True, False, False)
  chunk(63, 63 % 4, 63 % 3, 0, 0, 0, 0, False, False, False, False)

  wait_store(61 % 3)
  wait_store(62 % 3)
  wait_store(63 % 3)


def _pad(t):
  return jnp.pad(t, ((0, 0), (0, _DP - _D)))


_RN = 1024


def _combine_body(tok_ref, sums_ref, segf_ref, diff_ref, o_ref):
  o_ref[...] = (tok_ref[...] + sums_ref[:, :_D]
                + segf_ref[...] * diff_ref[...])


_combine = pl.pallas_call(
    _combine_body,
    out_shape=jax.ShapeDtypeStruct((_N, _D), jnp.float32),
    grid=(_N // _RN,),
    in_specs=[
        pl.BlockSpec((_RN, _D), lambda i: (i, 0)),
        pl.BlockSpec((_RN, _DP), lambda i: (i, 0)),
        pl.BlockSpec((_RN, 1), lambda i: (i, 0)),
        pl.BlockSpec((1, _D), lambda i: (0, 0)),
    ],
    out_specs=pl.BlockSpec((_RN, _D), lambda i: (i, 0)),
)


def kernel(token_vecs, pos_idx, seg_idx, col_idx, row_idx, rank_idx,
           pos, seg_id, col_id, row_id, rank_id):
  idx = jnp.stack([pos_idx.reshape(_N), seg_idx.reshape(_N),
                   col_idx.reshape(_N), row_idx.reshape(_N),
                   rank_idx.reshape(_N)], axis=0)
  idx_blocks = idx.reshape(5, _NBLK, _C).transpose(1, 0, 2)
  pos_eff = pos + seg_id[0][None, :]
  diff = (seg_id[1] - seg_id[0])[None, :]
  sums = _embed_sum(
      idx_blocks,
      _pad(pos_eff), _pad(col_id), _pad(row_id), _pad(rank_id))
  out = _combine(token_vecs.reshape(_N, _D), sums,
                 seg_idx.reshape(_N, 1).astype(jnp.float32), diff)
  return out.reshape(_B, _L, _D)
